# speedup vs baseline: 3.1211x; 3.1211x over previous
import functools

import jax
import jax.numpy as jnp
from jax import lax
from jax.experimental import pallas as pl
from jax.experimental.pallas import tpu as pltpu
from jax.experimental.pallas import tpu_sc as plsc

N = 10000
E = 320000
C = 128
NB = 64
NPAD = 10240

NC = 2
NS = 16
CW = 128
CHUNKS = 80
EPW = CHUNKS * CW
EPAD = NC * NS * EPW
ROWS_PER_TILE = NPAD // NS

_mesh = plsc.VectorSubcoreMesh(core_axis_name="c", subcore_axis_name="s")


@functools.partial(
    pl.kernel,
    out_type=jax.ShapeDtypeStruct((NC, NPAD, C), jnp.float32),
    mesh=_mesh,
    scratch_types=[
        pltpu.VMEM((CHUNKS, CW), jnp.int32),
        pltpu.VMEM((CHUNKS, CW), jnp.int32),
        pltpu.VMEM((CW, C), jnp.float32),
        pltpu.VMEM_SHARED((NPAD, C), jnp.float32),
        pltpu.SemaphoreType.DMA,
    ],
)
def _sc_segment_sum(h_hbm, src_hbm, dst_hbm, zero_hbm, out_hbm,
                    src_v, dst_v, buf, agg_sh, sem):
    c = lax.axis_index("c")
    s = lax.axis_index("s")
    row0 = s * ROWS_PER_TILE

    pltpu.sync_copy(src_hbm.at[c, s], src_v)
    pltpu.sync_copy(dst_hbm.at[c, s], dst_v)
    pltpu.sync_copy(zero_hbm.at[pl.ds(row0, ROWS_PER_TILE)],
                    agg_sh.at[pl.ds(row0, ROWS_PER_TILE)])
    plsc.subcore_barrier()

    def body(j, carry):
        pltpu.async_copy(h_hbm.at[src_v.at[j]], buf, sem).wait()
        pltpu.sync_copy(buf, agg_sh.at[dst_v.at[j]], add=True)
        return carry

    lax.fori_loop(0, CHUNKS, body, 0)

    plsc.subcore_barrier()
    pltpu.sync_copy(agg_sh.at[pl.ds(row0, ROWS_PER_TILE)],
                    out_hbm.at[c, pl.ds(row0, ROWS_PER_TILE)])


def _dot_t(a, w):
    return lax.dot_general(a, w, (((1,), (1,)), ((), ())),
                           preferred_element_type=jnp.float32)


BM = 512
GRID = NPAD // BM


def _layer1_body(agg_ref, x_ref, wrel_ref, b_ref, wroot_ref, out_ref):
    agg = agg_ref[0] + agg_ref[1]
    h = _dot_t(agg, wrel_ref[...]) + _dot_t(x_ref[...], wroot_ref[...])
    out_ref[...] = jnp.maximum(h + b_ref[...], 0.0)


def _tc_layer1(agg, x, wrel, b, wroot):
    return pl.pallas_call(
        _layer1_body,
        grid=(GRID,),
        in_specs=[
            pl.BlockSpec((NC, BM, C), lambda i: (0, i, 0)),
            pl.BlockSpec((BM, C), lambda i: (i, 0)),
            pl.BlockSpec((C, C), lambda i: (0, 0)),
            pl.BlockSpec((1, C), lambda i: (0, 0)),
            pl.BlockSpec((C, C), lambda i: (0, 0)),
        ],
        out_specs=pl.BlockSpec((BM, C), lambda i: (i, 0)),
        out_shape=jax.ShapeDtypeStruct((NPAD, C), jnp.float32),
    )(agg, x, wrel, b.reshape(1, C), wroot)


def _layer2_body(agg_ref, h1_ref, w2rel_ref, b2_ref, w2root_ref,
                 w3rel_ref, b3_ref, w3root_ref, p3_ref, q3_ref):
    agg = agg_ref[0] + agg_ref[1]
    h2 = _dot_t(agg, w2rel_ref[...]) + _dot_t(h1_ref[...], w2root_ref[...])
    h2 = jnp.maximum(h2 + b2_ref[...], 0.0)
    p3_ref[...] = _dot_t(h2, w3rel_ref[...])
    q3_ref[...] = _dot_t(h2, w3root_ref[...]) + b3_ref[...]


def _tc_layer2(agg, h1, w2rel, b2, w2root, w3rel, b3, w3root):
    return pl.pallas_call(
        _layer2_body,
        grid=(GRID,),
        in_specs=[
            pl.BlockSpec((NC, BM, C), lambda i: (0, i, 0)),
            pl.BlockSpec((BM, C), lambda i: (i, 0)),
            pl.BlockSpec((2 * C, C), lambda i: (0, 0)),
            pl.BlockSpec((1, 2 * C), lambda i: (0, 0)),
            pl.BlockSpec((2 * C, C), lambda i: (0, 0)),
            pl.BlockSpec((C, 2 * C), lambda i: (0, 0)),
            pl.BlockSpec((1, C), lambda i: (0, 0)),
            pl.BlockSpec((C, 2 * C), lambda i: (0, 0)),
        ],
        out_specs=[
            pl.BlockSpec((BM, C), lambda i: (i, 0)),
            pl.BlockSpec((BM, C), lambda i: (i, 0)),
        ],
        out_shape=[
            jax.ShapeDtypeStruct((NPAD, C), jnp.float32),
            jax.ShapeDtypeStruct((NPAD, C), jnp.float32),
        ],
    )(agg, h1, w2rel, b2.reshape(1, 2 * C), w2root,
      w3rel, b3.reshape(1, C), w3root)


def _layer3_body(agg_ref, q3_ref, batch_ref, wl1_ref, bl1_ref, wl2_ref,
                 bl2_ref, out_ref, pool_acc, cnt_acc):
    i = pl.program_id(0)

    @pl.when(i == 0)
    def _():
        pool_acc[...] = jnp.zeros_like(pool_acc)
        cnt_acc[...] = jnp.zeros_like(cnt_acc)

    h3 = agg_ref[0] + agg_ref[1] + q3_ref[...]
    bvec = batch_ref[0, 0, :]
    m = (lax.broadcasted_iota(jnp.int32, (NB, BM), 0)
         == bvec[None, :]).astype(jnp.float32)
    pool_acc[...] += lax.dot_general(m, h3, (((1,), (0,)), ((), ())),
                                     preferred_element_type=jnp.float32)
    cnt_acc[...] += jnp.broadcast_to(jnp.sum(m, axis=1, keepdims=True),
                                     (NB, C))

    @pl.when(i == GRID - 1)
    def _():
        g = pool_acc[...] / jnp.maximum(cnt_acc[...], 1.0)
        g = _dot_t(g, wl1_ref[...]) + bl1_ref[...]
        g = _dot_t(g, wl2_ref[...]) + bl2_ref[...]
        out_ref[...] = g


def _tc_layer3(agg, q3, batchp, wl1, bl1, wl2, bl2):
    return pl.pallas_call(
        _layer3_body,
        grid=(GRID,),
        in_specs=[
            pl.BlockSpec((NC, BM, C), lambda i: (0, i, 0)),
            pl.BlockSpec((BM, C), lambda i: (i, 0)),
            pl.BlockSpec((1, 1, BM), lambda i: (i, 0, 0)),
            pl.BlockSpec((NB, C), lambda i: (0, 0)),
            pl.BlockSpec((1, NB), lambda i: (0, 0)),
            pl.BlockSpec((16, NB), lambda i: (0, 0)),
            pl.BlockSpec((1, 16), lambda i: (0, 0)),
        ],
        out_specs=pl.BlockSpec((NB, 16), lambda i: (0, 0)),
        out_shape=jax.ShapeDtypeStruct((NB, 16), jnp.float32),
        scratch_shapes=[
            pltpu.VMEM((NB, C), jnp.float32),
            pltpu.VMEM((NB, C), jnp.float32),
        ],
    )(agg, q3, batchp, wl1, bl1.reshape(1, NB), wl2, bl2.reshape(1, 16))


def kernel(x, edge_index, batch, W1_rel, b1, W1_root, W2_rel, b2, W2_root,
           W3_rel, b3, W3_root, Wl1, bl1, Wl2, bl2):
    f32 = jnp.float32
    xp = jnp.pad(x.astype(f32), ((0, NPAD - N), (0, 0)))
    src = edge_index[0].astype(jnp.int32)
    dst = edge_index[1].astype(jnp.int32)
    srcp = jnp.pad(src, (0, EPAD - E)).reshape(NC, NS, CHUNKS, CW)
    dstp = jnp.pad(dst, (0, EPAD - E),
                   constant_values=NPAD - 1).reshape(NC, NS, CHUNKS, CW)
    batchp = jnp.pad(batch.astype(jnp.int32), (0, NPAD - N),
                     constant_values=NB).reshape(GRID, 1, BM)
    zeros_h = jnp.zeros((NPAD, C), f32)

    agg1 = _sc_segment_sum(xp, srcp, dstp, zeros_h)
    h1 = _tc_layer1(agg1, xp, W1_rel, b1, W1_root)
    agg2 = _sc_segment_sum(h1, srcp, dstp, zeros_h)
    p3, q3 = _tc_layer2(agg2, h1, W2_rel, b2, W2_root, W3_rel, b3, W3_root)
    agg3 = _sc_segment_sum(p3, srcp, dstp, zeros_h)
    return _tc_layer3(agg3, q3, batchp, Wl1, bl1, Wl2, bl2)
